# Initial kernel scaffold; baseline (speedup 1.0000x reference)
import jax
import jax.numpy as jnp
from jax.experimental import pallas as pl


def kernel(input, weight):
    raise NotImplementedError("write your pallas kernel here")


# speedup vs baseline: 2.1919x; 2.1919x over previous
import functools

import jax
import jax.numpy as jnp
from jax import lax
from jax.experimental import pallas as pl
from jax.experimental.pallas import tpu as pltpu
from jax.experimental.pallas import tpu_sc as plsc

DIM = 1024
NUM_CORES = 2
NUM_SUBCORES = 16
NUM_WORKERS = NUM_CORES * NUM_SUBCORES
CHUNK = 64


@functools.partial(jax.jit, static_argnames=("total",))
def _gather_rows(idx, weight, *, total):
    rows_per_w = total // NUM_WORKERS
    n_chunks = rows_per_w // CHUNK
    mesh = plsc.VectorSubcoreMesh(core_axis_name="c", subcore_axis_name="s")

    @functools.partial(
        pl.kernel,
        out_type=jax.ShapeDtypeStruct((total, DIM), jnp.float32),
        mesh=mesh,
        scratch_types=[
            pltpu.VMEM((rows_per_w,), jnp.int32),
            pltpu.VMEM((CHUNK, DIM), jnp.float32),
            pltpu.SemaphoreType.DMA,
        ],
    )
    def k(idx_hbm, table_hbm, out_hbm, idx_v, buf, sem):
        wid = lax.axis_index("s") * NUM_CORES + lax.axis_index("c")
        base = wid * rows_per_w
        pltpu.sync_copy(idx_hbm.at[pl.ds(base, rows_per_w)], idx_v)

        def body(j, carry):
            off = j * CHUNK
            pltpu.async_copy(
                table_hbm.at[idx_v.at[pl.ds(off, CHUNK)]], buf, sem
            ).wait()
            pltpu.sync_copy(buf, out_hbm.at[pl.ds(base + off, CHUNK)])
            return carry

        lax.fori_loop(0, n_chunks, body, 0)

    return k(idx, weight)


def kernel(input, weight):
    total = input.shape[0] * input.shape[1]
    idx = input.reshape(total).astype(jnp.int32)
    out = _gather_rows(idx, weight, total=total)
    return out.reshape(input.shape + (DIM,))


# speedup vs baseline: 2.3841x; 1.0877x over previous
import functools

import jax
import jax.numpy as jnp
from jax import lax
from jax.experimental import pallas as pl
from jax.experimental.pallas import tpu as pltpu
from jax.experimental.pallas import tpu_sc as plsc

DIM = 1024
NUM_CORES = 2
NUM_SUBCORES = 16
NUM_WORKERS = NUM_CORES * NUM_SUBCORES
CHUNK = 16
NBUF = 4
LOOK = 2


@functools.partial(jax.jit, static_argnames=("total",))
def _gather_rows(idx, weight, *, total):
    rows_per_w = total // NUM_WORKERS
    n_chunks = rows_per_w // CHUNK
    n_outer = n_chunks // NBUF
    mesh = plsc.VectorSubcoreMesh(core_axis_name="c", subcore_axis_name="s")

    @functools.partial(
        pl.kernel,
        out_type=jax.ShapeDtypeStruct((total, DIM), jnp.float32),
        mesh=mesh,
        scratch_types=[
            pltpu.VMEM((rows_per_w,), jnp.int32),
            pltpu.VMEM((NBUF, CHUNK, DIM), jnp.float32),
            [pltpu.SemaphoreType.DMA] * NBUF,
            [pltpu.SemaphoreType.DMA] * NBUF,
        ],
    )
    def k(idx_hbm, table_hbm, out_hbm, idx_v, bufs, gsem, wsem):
        wid = lax.axis_index("s") * NUM_CORES + lax.axis_index("c")
        base = wid * rows_per_w
        pltpu.sync_copy(idx_hbm.at[pl.ds(base, rows_per_w)], idx_v)

        def start_gather(g, b):
            pltpu.make_async_copy(
                table_hbm.at[idx_v.at[pl.ds(g * CHUNK, CHUNK)]],
                bufs.at[b],
                gsem[b],
            ).start()

        def wait_gather(b):
            pltpu.make_async_copy(
                table_hbm.at[idx_v.at[pl.ds(0, CHUNK)]], bufs.at[b], gsem[b]
            ).wait()

        def start_write(j, b):
            pltpu.make_async_copy(
                bufs.at[b], out_hbm.at[pl.ds(base + j * CHUNK, CHUNK)], wsem[b]
            ).start()

        def wait_write(b):
            pltpu.make_async_copy(
                bufs.at[b], out_hbm.at[pl.ds(base, CHUNK)], wsem[b]
            ).wait()

        for c in range(LOOK):
            start_gather(c, c)

        def outer(o, carry):
            for b in range(NBUF):
                j = o * NBUF + b
                g = j + LOOK
                gb = (b + LOOK) % NBUF

                @pl.when(g < n_chunks)
                def _issue():
                    @pl.when(g >= NBUF)
                    def _drain():
                        wait_write(gb)

                    start_gather(g, gb)

                wait_gather(b)
                start_write(j, b)
            return carry

        lax.fori_loop(0, n_outer, outer, 0)
        for b in range(NBUF):
            wait_write(b)

    return k(idx, weight)


def kernel(input, weight):
    total = input.shape[0] * input.shape[1]
    idx = input.reshape(total).astype(jnp.int32)
    out = _gather_rows(idx, weight, total=total)
    return out.reshape(input.shape + (DIM,))


# speedup vs baseline: 2.4064x; 1.0094x over previous
import functools

import jax
import jax.numpy as jnp
from jax import lax
from jax.experimental import pallas as pl
from jax.experimental.pallas import tpu as pltpu
from jax.experimental.pallas import tpu_sc as plsc

DIM = 1024
NUM_CORES = 2
NUM_SUBCORES = 16
NUM_WORKERS = NUM_CORES * NUM_SUBCORES
CHUNK = 8
NBUF = 8
LOOK = 4


@functools.partial(jax.jit, static_argnames=("total",))
def _gather_rows(idx, weight, *, total):
    rows_per_w = total // NUM_WORKERS
    n_chunks = rows_per_w // CHUNK
    n_outer = n_chunks // NBUF
    mesh = plsc.VectorSubcoreMesh(core_axis_name="c", subcore_axis_name="s")

    @functools.partial(
        pl.kernel,
        out_type=jax.ShapeDtypeStruct((total, DIM), jnp.float32),
        mesh=mesh,
        scratch_types=[
            pltpu.VMEM((rows_per_w,), jnp.int32),
            pltpu.VMEM((NBUF, CHUNK, DIM), jnp.float32),
            [pltpu.SemaphoreType.DMA] * NBUF,
            [pltpu.SemaphoreType.DMA] * NBUF,
        ],
    )
    def k(idx_hbm, table_hbm, out_hbm, idx_v, bufs, gsem, wsem):
        wid = lax.axis_index("s") * NUM_CORES + lax.axis_index("c")
        base = wid * rows_per_w
        pltpu.sync_copy(idx_hbm.at[pl.ds(base, rows_per_w)], idx_v)

        def start_gather(g, b):
            pltpu.make_async_copy(
                table_hbm.at[idx_v.at[pl.ds(g * CHUNK, CHUNK)]],
                bufs.at[b],
                gsem[b],
            ).start()

        def wait_gather(b):
            pltpu.make_async_copy(
                table_hbm.at[idx_v.at[pl.ds(0, CHUNK)]], bufs.at[b], gsem[b]
            ).wait()

        def start_write(j, b):
            pltpu.make_async_copy(
                bufs.at[b], out_hbm.at[pl.ds(base + j * CHUNK, CHUNK)], wsem[b]
            ).start()

        def wait_write(b):
            pltpu.make_async_copy(
                bufs.at[b], out_hbm.at[pl.ds(base, CHUNK)], wsem[b]
            ).wait()

        for c in range(LOOK):
            start_gather(c, c)

        def outer(o, carry):
            for b in range(NBUF):
                j = o * NBUF + b
                g = j + LOOK
                gb = (b + LOOK) % NBUF

                @pl.when(g < n_chunks)
                def _issue():
                    @pl.when(g >= NBUF)
                    def _drain():
                        wait_write(gb)

                    start_gather(g, gb)

                wait_gather(b)
                start_write(j, b)
            return carry

        lax.fori_loop(0, n_outer, outer, 0)
        for b in range(NBUF):
            wait_write(b)

    return k(idx, weight)


def kernel(input, weight):
    total = input.shape[0] * input.shape[1]
    idx = input.reshape(total).astype(jnp.int32)
    out = _gather_rows(idx, weight, total=total)
    return out.reshape(input.shape + (DIM,))
